# Initial kernel scaffold; baseline (speedup 1.0000x reference)
import jax
import jax.numpy as jnp
from jax.experimental import pallas as pl


def kernel(hidden_states, gate_w, e_score_correction_bias, w1, w2, shared_w1, shared_w2):
    raise NotImplementedError("write your pallas kernel here")


# speedup vs baseline: 3.5626x; 3.5626x over previous
import functools

import jax
import jax.numpy as jnp
from jax import lax
from jax.experimental import pallas as pl
from jax.experimental.pallas import tpu as pltpu

E = 64
K = 2
D = 1024
I = 512
RSF = 2.5
T = 4096

TB = 128
NB = 128
NROWS = NB * TB


def _gate_body(x_ref, gw_ref, b_ref, idx_ref, w_ref):
    x = x_ref[...]
    logits = lax.dot_general(x, gw_ref[...], (((1,), (1,)), ((), ())),
                             preferred_element_type=jnp.float32)
    scores = jax.nn.sigmoid(logits)
    sc = scores + b_ref[...]
    iota = lax.broadcasted_iota(jnp.int32, sc.shape, 1)
    m1 = jnp.max(sc, axis=1, keepdims=True)
    idx1 = jnp.min(jnp.where(sc >= m1, iota, E), axis=1, keepdims=True)
    not1 = iota != idx1
    m2 = jnp.max(jnp.where(not1, sc, -jnp.inf), axis=1, keepdims=True)
    idx2 = jnp.min(jnp.where((sc >= m2) & not1, iota, E), axis=1, keepdims=True)
    w1v = jnp.sum(jnp.where(iota == idx1, scores, 0.0), axis=1, keepdims=True)
    w2v = jnp.sum(jnp.where(iota == idx2, scores, 0.0), axis=1, keepdims=True)
    s = w1v + w2v
    idx_ref[...] = jnp.concatenate([idx1, idx2], axis=1)
    w_ref[...] = jnp.concatenate([w1v / s * RSF, w2v / s * RSF], axis=1)


def _gating(x, gate_w, bias):
    TBA = 512
    return pl.pallas_call(
        _gate_body,
        grid=(T // TBA,),
        in_specs=[
            pl.BlockSpec((TBA, D), lambda i: (i, 0)),
            pl.BlockSpec((E, D), lambda i: (0, 0)),
            pl.BlockSpec((1, E), lambda i: (0, 0)),
        ],
        out_specs=[
            pl.BlockSpec((TBA, K), lambda i: (i, 0)),
            pl.BlockSpec((TBA, K), lambda i: (i, 0)),
        ],
        out_shape=[
            jax.ShapeDtypeStruct((T, K), jnp.int32),
            jax.ShapeDtypeStruct((T, K), jnp.float32),
        ],
    )(x, gate_w, bias.reshape(1, E))


def _moe_body(be_ref, bx_ref, x_ref, w1_ref, w2_ref, y_ref):
    x = x_ref[...]
    gu = lax.dot_general(x, w1_ref[0], (((1,), (1,)), ((), ())),
                         preferred_element_type=jnp.float32)
    g = gu[:, :I]
    u = gu[:, I:]
    h = g * jax.nn.sigmoid(g) * u
    y_ref[...] = lax.dot_general(h, w2_ref[0], (((1,), (1,)), ((), ())),
                                 preferred_element_type=jnp.float32)


def _grouped_mlp(x_sorted, w1, w2, block_expert, bx):
    grid_spec = pltpu.PrefetchScalarGridSpec(
        num_scalar_prefetch=2,
        grid=(NB,),
        in_specs=[
            pl.BlockSpec((TB, D), lambda i, be, bx: (bx[i], 0)),
            pl.BlockSpec((1, 2 * I, D), lambda i, be, bx: (be[i], 0, 0)),
            pl.BlockSpec((1, D, I), lambda i, be, bx: (be[i], 0, 0)),
        ],
        out_specs=pl.BlockSpec((TB, D), lambda i, be, bx: (bx[i], 0)),
    )
    return pl.pallas_call(
        _moe_body,
        grid_spec=grid_spec,
        out_shape=jax.ShapeDtypeStruct((NROWS, D), jnp.float32),
    )(block_expert, bx, x_sorted, w1, w2)


def _shared_body(x_ref, w1_ref, w2_ref, y_ref):
    x = x_ref[...]
    gu = lax.dot_general(x, w1_ref[...], (((1,), (1,)), ((), ())),
                         preferred_element_type=jnp.float32)
    g = gu[:, :I]
    u = gu[:, I:]
    h = g * jax.nn.sigmoid(g) * u
    y_ref[...] = lax.dot_general(h, w2_ref[...], (((1,), (1,)), ((), ())),
                                 preferred_element_type=jnp.float32)


def _shared_mlp(x, shared_w1, shared_w2):
    TBS = 512
    return pl.pallas_call(
        _shared_body,
        grid=(T // TBS,),
        in_specs=[
            pl.BlockSpec((TBS, D), lambda i: (i, 0)),
            pl.BlockSpec((2 * I, D), lambda i: (0, 0)),
            pl.BlockSpec((D, I), lambda i: (0, 0)),
        ],
        out_specs=pl.BlockSpec((TBS, D), lambda i: (i, 0)),
        out_shape=jax.ShapeDtypeStruct((T, D), jnp.float32),
    )(x, shared_w1, shared_w2)


def kernel(hidden_states, gate_w, e_score_correction_bias, w1, w2,
           shared_w1, shared_w2):
    x = hidden_states
    topk_idx, topk_w = _gating(x, gate_w, e_score_correction_bias)

    flat_e = topk_idx.reshape(-1)
    cnt = jnp.zeros((E,), jnp.int32).at[flat_e].add(1)
    nb = (cnt + (TB - 1)) // TB
    raw_off = jnp.cumsum(cnt) - cnt
    pad_off = (jnp.cumsum(nb) - nb) * TB
    order = jnp.argsort(flat_e, stable=True)
    e_sorted = flat_e[order]
    pos_sorted = pad_off[e_sorted] + (jnp.arange(T * K, dtype=jnp.int32)
                                      - raw_off[e_sorted])
    pos = jnp.zeros((T * K,), jnp.int32).at[order].set(pos_sorted)
    x_sorted = jnp.zeros((NROWS, D), x.dtype).at[pos].set(
        jnp.repeat(x, K, axis=0))
    used = jnp.sum(nb)
    blk = jnp.arange(NB, dtype=jnp.int32)
    block_expert = jnp.minimum(
        jnp.searchsorted(jnp.cumsum(nb), blk, side='right'), E - 1
    ).astype(jnp.int32)
    bx = jnp.where(blk < used, blk, used - 1).astype(jnp.int32)

    y_sorted = _grouped_mlp(x_sorted, w1, w2, block_expert, bx)
    shared_out = _shared_mlp(x, shared_w1, shared_w2)

    pos2 = pos.reshape(T, K)
    out = (shared_out
           + topk_w[:, 0:1] * y_sorted[pos2[:, 0]]
           + topk_w[:, 1:2] * y_sorted[pos2[:, 1]])
    return out


# speedup vs baseline: 5.9926x; 1.6821x over previous
import functools

import jax
import jax.numpy as jnp
from jax import lax
from jax.experimental import pallas as pl
from jax.experimental.pallas import tpu as pltpu
from jax.experimental.pallas import tpu_sc as plsc

E = 64
K = 2
D = 1024
I = 512
RSF = 2.5
T = 4096

TB = 128
NB = 128
NROWS = NB * TB
NW = 32
TPW = T // NW


def _gate_body(x_ref, gw_ref, b_ref, idx_ref, w_ref, h_ref):
    x = x_ref[...]
    logits = lax.dot_general(x, gw_ref[...], (((1,), (1,)), ((), ())),
                             preferred_element_type=jnp.float32)
    scores = jax.nn.sigmoid(logits)
    sc = scores + b_ref[...]
    iota = lax.broadcasted_iota(jnp.int32, sc.shape, 1)
    m1 = jnp.max(sc, axis=1, keepdims=True)
    idx1 = jnp.min(jnp.where(sc >= m1, iota, E), axis=1, keepdims=True)
    not1 = iota != idx1
    m2 = jnp.max(jnp.where(not1, sc, -jnp.inf), axis=1, keepdims=True)
    idx2 = jnp.min(jnp.where((sc >= m2) & not1, iota, E), axis=1, keepdims=True)
    w1v = jnp.sum(jnp.where(iota == idx1, scores, 0.0), axis=1, keepdims=True)
    w2v = jnp.sum(jnp.where(iota == idx2, scores, 0.0), axis=1, keepdims=True)
    s = w1v + w2v
    idx_ref[...] = jnp.concatenate([idx1, idx2], axis=1)
    w_ref[...] = jnp.concatenate([w1v / s * RSF, w2v / s * RSF], axis=1)
    oh = ((iota == idx1) | (iota == idx2)).astype(jnp.int32)
    subs = [jnp.sum(oh[i * TPW:(i + 1) * TPW], axis=0, keepdims=True)
            for i in range(oh.shape[0] // TPW)]
    h_ref[0] = jnp.concatenate(subs, axis=0)


def _gating(x, gate_w, bias):
    TBA = 512
    return pl.pallas_call(
        _gate_body,
        grid=(T // TBA,),
        in_specs=[
            pl.BlockSpec((TBA, D), lambda i: (i, 0)),
            pl.BlockSpec((E, D), lambda i: (0, 0)),
            pl.BlockSpec((1, E), lambda i: (0, 0)),
        ],
        out_specs=[
            pl.BlockSpec((TBA, K), lambda i: (i, 0)),
            pl.BlockSpec((TBA, K), lambda i: (i, 0)),
            pl.BlockSpec((1, TBA // TPW, E), lambda i: (i, 0, 0)),
        ],
        out_shape=[
            jax.ShapeDtypeStruct((T, K), jnp.int32),
            jax.ShapeDtypeStruct((T, K), jnp.float32),
            jax.ShapeDtypeStruct((T // TBA, TBA // TPW, E), jnp.int32),
        ],
    )(x, gate_w, bias.reshape(1, E))


_LANES = lambda: lax.broadcasted_iota(jnp.int32, (16,), 0)


def _vec16(read, base):
    lanes = _LANES()
    v = jnp.zeros((16,), jnp.int32)
    for i in range(16):
        v = jnp.where(lanes == i, read(base + i), v)
    return v


def _dispatch_sc(topk_idx, hist32, x):
    mesh = plsc.VectorSubcoreMesh(core_axis_name="c", subcore_axis_name="s")

    @functools.partial(
        pl.kernel, mesh=mesh,
        out_type=[
            jax.ShapeDtypeStruct((NW, K * TPW // 64, 64), jnp.int32),
            jax.ShapeDtypeStruct((NROWS, D), jnp.float32),
            jax.ShapeDtypeStruct((NB,), jnp.int32),
            jax.ShapeDtypeStruct((NB,), jnp.int32),
        ],
        scratch_types=[
            pltpu.VMEM((TPW * K,), jnp.int32),
            pltpu.VMEM((NW, E), jnp.int32),
            pltpu.VMEM((E,), jnp.int32),
            pltpu.SMEM((E,), jnp.int32),
            pltpu.SMEM((K, TPW), jnp.int32),
            pltpu.VMEM((K * TPW // 64, 64), jnp.int32),
            pltpu.VMEM((64, D), jnp.float32),
            pltpu.SMEM((NB,), jnp.int32),
            pltpu.SMEM((NB,), jnp.int32),
            pltpu.VMEM((NB,), jnp.int32),
            pltpu.VMEM((NB,), jnp.int32),
            pltpu.SemaphoreType.DMA,
        ],
    )
    def k(tki, hist_hbm, x_hbm, pos_hbm, xs_hbm, be_hbm, bx_hbm,
          eidx, hist, nbv, cnt, posbuf, posv, xbuf,
          bebuf, bxbuf, bev, bxv, sem):
        wid = lax.axis_index("s") * 2 + lax.axis_index("c")
        pltpu.sync_copy(tki.at[pl.ds(wid * TPW * K, TPW * K)], eidx)
        pltpu.sync_copy(hist_hbm, hist)

        acc = jnp.int32(0)
        for i in range(E // 16):
            sl = pl.ds(i * 16, 16)
            tot = lax.fori_loop(
                0, NW, lambda w2, a: a + hist[w2, sl],
                jnp.zeros((16,), jnp.int32))
            nb16 = lax.shift_right_logical(tot + (TB - 1), 7)
            nbv[sl] = nb16
            pre16 = lax.fori_loop(
                0, wid, lambda w2, a: a + hist[w2, sl],
                jnp.zeros((16,), jnp.int32))
            pad16 = nb16 * TB
            for l in range(16):
                cnt[i * 16 + l] = acc + pre16[l]
                acc = acc + pad16[l]

        def pos_body(c, _):
            ev = eidx[pl.ds(c * 16, 16)]
            for i in range(16):
                e = ev[i]
                p = cnt[e]
                cnt[e] = p + 1
                posbuf[i % 2, c * 8 + i // 2] = p
            return 0
        lax.fori_loop(0, TPW * K // 16, pos_body, 0)
        for kk in range(K):
            for c in range(TPW // 16):
                posv[kk * 2 + c // 4, pl.ds((c % 4) * 16, 16)] = _vec16(
                    lambda i: posbuf[kk, i], c * 16)
        pltpu.sync_copy(posv, pos_hbm.at[wid])

        for c in range(TPW // 64):
            pltpu.sync_copy(x_hbm.at[pl.ds(wid * TPW + c * 64, 64)], xbuf)
            for kk in range(K):
                pltpu.async_copy(
                    xbuf, xs_hbm.at[posv.at[kk * 2 + c]], sem).wait()

        @pl.when(wid == 0)
        def _():
            cur = jnp.int32(0)
            for e in range(E):
                nb16 = nbv[pl.ds((e // 16) * 16, 16)]
                nbe = nb16[e % 16]
                def inner(b, _, e=e, cur=cur):
                    bebuf[cur + b] = e
                    bxbuf[cur + b] = cur + b
                    return 0
                lax.fori_loop(0, nbe, inner, 0)
                cur = cur + nbe
            used = cur
            lastbe = bebuf[used - 1]
            def tail(i, _):
                bebuf[i] = lastbe
                bxbuf[i] = used - 1
                return 0
            lax.fori_loop(used, NB, tail, 0)
            for c in range(NB // 16):
                bev[pl.ds(c * 16, 16)] = _vec16(lambda i: bebuf[i], c * 16)
                bxv[pl.ds(c * 16, 16)] = _vec16(lambda i: bxbuf[i], c * 16)
            pltpu.sync_copy(bev, be_hbm)
            pltpu.sync_copy(bxv, bx_hbm)

    return k(topk_idx.reshape(T * K), hist32, x)


def _moe_body(be_ref, bx_ref, x_ref, w1_ref, w2_ref, y_ref):
    x = x_ref[...]
    gu = lax.dot_general(x, w1_ref[0], (((1,), (1,)), ((), ())),
                         preferred_element_type=jnp.float32)
    g = gu[:, :I]
    u = gu[:, I:]
    h = g * jax.nn.sigmoid(g) * u
    y_ref[...] = lax.dot_general(h, w2_ref[0], (((1,), (1,)), ((), ())),
                                 preferred_element_type=jnp.float32)


def _grouped_mlp(x_sorted, w1, w2, block_expert, bx):
    grid_spec = pltpu.PrefetchScalarGridSpec(
        num_scalar_prefetch=2,
        grid=(NB,),
        in_specs=[
            pl.BlockSpec((TB, D), lambda i, be, bx: (bx[i], 0)),
            pl.BlockSpec((1, 2 * I, D), lambda i, be, bx: (be[i], 0, 0)),
            pl.BlockSpec((1, D, I), lambda i, be, bx: (be[i], 0, 0)),
        ],
        out_specs=pl.BlockSpec((TB, D), lambda i, be, bx: (bx[i], 0)),
    )
    return pl.pallas_call(
        _moe_body,
        grid_spec=grid_spec,
        out_shape=jax.ShapeDtypeStruct((NROWS, D), jnp.float32),
    )(block_expert, bx, x_sorted, w1, w2)


def _shared_body(x_ref, w1_ref, w2_ref, y_ref):
    x = x_ref[...]
    gu = lax.dot_general(x, w1_ref[...], (((1,), (1,)), ((), ())),
                         preferred_element_type=jnp.float32)
    g = gu[:, :I]
    u = gu[:, I:]
    h = g * jax.nn.sigmoid(g) * u
    y_ref[...] = lax.dot_general(h, w2_ref[...], (((1,), (1,)), ((), ())),
                                 preferred_element_type=jnp.float32)


def _shared_mlp(x, shared_w1, shared_w2):
    TBS = 512
    return pl.pallas_call(
        _shared_body,
        grid=(T // TBS,),
        in_specs=[
            pl.BlockSpec((TBS, D), lambda i: (i, 0)),
            pl.BlockSpec((2 * I, D), lambda i: (0, 0)),
            pl.BlockSpec((D, I), lambda i: (0, 0)),
        ],
        out_specs=pl.BlockSpec((TBS, D), lambda i: (i, 0)),
        out_shape=jax.ShapeDtypeStruct((T, D), jnp.float32),
    )(x, shared_w1, shared_w2)


def _combine_sc(pos, topk_w, y_sorted, shared_out):
    mesh = plsc.VectorSubcoreMesh(core_axis_name="c", subcore_axis_name="s")
    CH = 32

    @functools.partial(
        pl.kernel, mesh=mesh,
        out_type=jax.ShapeDtypeStruct((T, D), jnp.float32),
        scratch_types=[
            pltpu.VMEM((K * TPW // 64, 64), jnp.int32),
            pltpu.VMEM((TPW * K,), jnp.float32),
            pltpu.VMEM((CH, D), jnp.float32),
            pltpu.VMEM((CH, D), jnp.float32),
            pltpu.VMEM((CH, D), jnp.float32),
            pltpu.SemaphoreType.DMA,
            pltpu.SemaphoreType.DMA,
        ],
    )
    def k(pos_hbm, w_hbm, ys_hbm, sh_hbm, out_hbm,
          posv, wv, buf0, buf1, acc, sem0, sem1):
        wid = lax.axis_index("s") * 2 + lax.axis_index("c")
        pltpu.sync_copy(pos_hbm.at[wid], posv)
        pltpu.sync_copy(w_hbm.at[pl.ds(wid * TPW * K, TPW * K)], wv)
        for c in range(TPW // CH):
            base = wid * TPW + c * CH
            i0 = posv.at[c // 2, pl.ds((c % 2) * CH, CH)]
            i1 = posv.at[2 + c // 2, pl.ds((c % 2) * CH, CH)]
            cp0 = pltpu.async_copy(ys_hbm.at[i0], buf0, sem0)
            cp1 = pltpu.async_copy(ys_hbm.at[i1], buf1, sem1)
            pltpu.sync_copy(sh_hbm.at[pl.ds(base, CH)], acc)
            cp0.wait()
            cp1.wait()

            for cc in range(CH // 8):
                wvec = wv[pl.ds(c * CH * K + cc * 16, 16)]
                for i in range(8):
                    j = cc * 8 + i
                    w0 = wvec[2 * i]
                    w1s = wvec[2 * i + 1]
                    def vbody(v, _, j=j, w0=w0, w1s=w1s):
                        sl = pl.ds(v * 16, 16)
                        acc[j, sl] = (acc[j, sl] + w0 * buf0[j, sl]
                                      + w1s * buf1[j, sl])
                        return 0
                    lax.fori_loop(0, D // 16, vbody, 0)
            pltpu.sync_copy(acc, out_hbm.at[pl.ds(base, CH)])

    return k(pos, topk_w.reshape(T * K), y_sorted, shared_out)


def kernel(hidden_states, gate_w, e_score_correction_bias, w1, w2,
           shared_w1, shared_w2):
    x = hidden_states
    topk_idx, topk_w, hist = _gating(x, gate_w, e_score_correction_bias)
    pos, x_sorted, block_expert, bx = _dispatch_sc(
        topk_idx, hist.reshape(NW, E), x)
    y_sorted = _grouped_mlp(x_sorted, w1, w2, block_expert, bx)
    shared_out = _shared_mlp(x, shared_w1, shared_w2)
    return _combine_sc(pos, topk_w, y_sorted, shared_out)


# speedup vs baseline: 6.2288x; 1.0394x over previous
import functools

import jax
import jax.numpy as jnp
from jax import lax
from jax.experimental import pallas as pl
from jax.experimental.pallas import tpu as pltpu
from jax.experimental.pallas import tpu_sc as plsc

E = 64
K = 2
D = 1024
I = 512
RSF = 2.5
T = 4096

TB = 128
NB = 128
NROWS = NB * TB
NW = 32
TPW = T // NW


def _gate_body(x_ref, gw_ref, b_ref, idx_ref, w_ref, h_ref):
    x = x_ref[...]
    logits = lax.dot_general(x, gw_ref[...], (((1,), (1,)), ((), ())),
                             preferred_element_type=jnp.float32)
    scores = jax.nn.sigmoid(logits)
    sc = scores + b_ref[...]
    iota = lax.broadcasted_iota(jnp.int32, sc.shape, 1)
    m1 = jnp.max(sc, axis=1, keepdims=True)
    idx1 = jnp.min(jnp.where(sc >= m1, iota, E), axis=1, keepdims=True)
    not1 = iota != idx1
    m2 = jnp.max(jnp.where(not1, sc, -jnp.inf), axis=1, keepdims=True)
    idx2 = jnp.min(jnp.where((sc >= m2) & not1, iota, E), axis=1, keepdims=True)
    w1v = jnp.sum(jnp.where(iota == idx1, scores, 0.0), axis=1, keepdims=True)
    w2v = jnp.sum(jnp.where(iota == idx2, scores, 0.0), axis=1, keepdims=True)
    s = w1v + w2v
    idx_ref[...] = jnp.concatenate([idx1, idx2], axis=1)
    w_ref[...] = jnp.concatenate(
        [jnp.transpose(w1v / s * RSF), jnp.transpose(w2v / s * RSF)], axis=0)
    oh = ((iota == idx1) | (iota == idx2)).astype(jnp.int32)
    subs = [jnp.sum(oh[i * TPW:(i + 1) * TPW], axis=0, keepdims=True)
            for i in range(oh.shape[0] // TPW)]
    h_ref[0] = jnp.concatenate(subs, axis=0)


def _gating(x, gate_w, bias):
    TBA = 512
    return pl.pallas_call(
        _gate_body,
        grid=(T // TBA,),
        in_specs=[
            pl.BlockSpec((TBA, D), lambda i: (i, 0)),
            pl.BlockSpec((E, D), lambda i: (0, 0)),
            pl.BlockSpec((1, E), lambda i: (0, 0)),
        ],
        out_specs=[
            pl.BlockSpec((TBA, K), lambda i: (i, 0)),
            pl.BlockSpec((K, TBA), lambda i: (0, i)),
            pl.BlockSpec((1, TBA // TPW, E), lambda i: (i, 0, 0)),
        ],
        out_shape=[
            jax.ShapeDtypeStruct((T, K), jnp.int32),
            jax.ShapeDtypeStruct((K, T), jnp.float32),
            jax.ShapeDtypeStruct((T // TBA, TBA // TPW, E), jnp.int32),
        ],
    )(x, gate_w, bias.reshape(1, E))


_LANES = lambda: lax.broadcasted_iota(jnp.int32, (16,), 0)


def _vec16(read, base):
    lanes = _LANES()
    v = jnp.zeros((16,), jnp.int32)
    for i in range(16):
        v = jnp.where(lanes == i, read(base + i), v)
    return v


def _dispatch_sc(topk_idx, topk_wT, hist32, x):
    mesh = plsc.VectorSubcoreMesh(core_axis_name="c", subcore_axis_name="s")

    @functools.partial(
        pl.kernel, mesh=mesh,
        out_type=[
            jax.ShapeDtypeStruct((NW, K * TPW // 64, 64), jnp.int32),
            jax.ShapeDtypeStruct((NROWS, D), jnp.float32),
            jax.ShapeDtypeStruct((NROWS,), jnp.float32),
            jax.ShapeDtypeStruct((NB,), jnp.int32),
            jax.ShapeDtypeStruct((NB,), jnp.int32),
        ],
        scratch_types=[
            pltpu.VMEM((TPW * K,), jnp.int32),
            pltpu.VMEM((NW, E), jnp.int32),
            pltpu.VMEM((E,), jnp.int32),
            pltpu.SMEM((E,), jnp.int32),
            pltpu.SMEM((K, TPW), jnp.int32),
            pltpu.VMEM((K * TPW // 64, 64), jnp.int32),
            pltpu.VMEM((K * TPW // 64, 64), jnp.float32),
            pltpu.VMEM((64, D), jnp.float32),
            pltpu.SMEM((NB,), jnp.int32),
            pltpu.SMEM((NB,), jnp.int32),
            pltpu.VMEM((NB,), jnp.int32),
            pltpu.VMEM((NB,), jnp.int32),
            pltpu.SemaphoreType.DMA,
        ],
    )
    def k(tki, wT, hist_hbm, x_hbm, pos_hbm, xs_hbm, ws_hbm, be_hbm, bx_hbm,
          eidx, hist, nbv, cnt, posbuf, posv, wbuf, xbuf,
          bebuf, bxbuf, bev, bxv, sem):
        wid = lax.axis_index("s") * 2 + lax.axis_index("c")
        pltpu.sync_copy(tki.at[pl.ds(wid * TPW * K, TPW * K)], eidx)
        pltpu.sync_copy(hist_hbm, hist)
        for kk in range(K):
            for c in range(TPW // 64):
                pltpu.sync_copy(
                    wT.at[kk, pl.ds(wid * TPW + c * 64, 64)],
                    wbuf.at[kk * 2 + c])

        acc = jnp.int32(0)
        for i in range(E // 16):
            sl = pl.ds(i * 16, 16)
            tot = lax.fori_loop(
                0, NW, lambda w2, a: a + hist[w2, sl],
                jnp.zeros((16,), jnp.int32))
            nb16 = lax.shift_right_logical(tot + (TB - 1), 7)
            nbv[sl] = nb16
            pre16 = lax.fori_loop(
                0, wid, lambda w2, a: a + hist[w2, sl],
                jnp.zeros((16,), jnp.int32))
            pad16 = nb16 * TB
            for l in range(16):
                cnt[i * 16 + l] = acc + pre16[l]
                acc = acc + pad16[l]

        def pos_body(c, _):
            ev = eidx[pl.ds(c * 16, 16)]
            for i in range(16):
                e = ev[i]
                p = cnt[e]
                cnt[e] = p + 1
                posbuf[i % 2, c * 8 + i // 2] = p
            return 0
        lax.fori_loop(0, TPW * K // 16, pos_body, 0)
        for kk in range(K):
            for c in range(TPW // 16):
                posv[kk * 2 + c // 4, pl.ds((c % 4) * 16, 16)] = _vec16(
                    lambda i: posbuf[kk, i], c * 16)
        pltpu.sync_copy(posv, pos_hbm.at[wid])

        for r in range(K * TPW // 64):
            pltpu.async_copy(wbuf.at[r], ws_hbm.at[posv.at[r]], sem).wait()
        for c in range(TPW // 64):
            pltpu.sync_copy(x_hbm.at[pl.ds(wid * TPW + c * 64, 64)], xbuf)
            for kk in range(K):
                pltpu.async_copy(
                    xbuf, xs_hbm.at[posv.at[kk * 2 + c]], sem).wait()

        @pl.when(wid == 0)
        def _():
            cur = jnp.int32(0)
            for e in range(E):
                nb16 = nbv[pl.ds((e // 16) * 16, 16)]
                nbe = nb16[e % 16]
                def inner(b, _, e=e, cur=cur):
                    bebuf[cur + b] = e
                    bxbuf[cur + b] = cur + b
                    return 0
                lax.fori_loop(0, nbe, inner, 0)
                cur = cur + nbe
            used = cur
            lastbe = bebuf[used - 1]
            def tail(i, _):
                bebuf[i] = lastbe
                bxbuf[i] = used - 1
                return 0
            lax.fori_loop(used, NB, tail, 0)
            for c in range(NB // 16):
                bev[pl.ds(c * 16, 16)] = _vec16(lambda i: bebuf[i], c * 16)
                bxv[pl.ds(c * 16, 16)] = _vec16(lambda i: bxbuf[i], c * 16)
            pltpu.sync_copy(bev, be_hbm)
            pltpu.sync_copy(bxv, bx_hbm)

    return k(topk_idx.reshape(T * K), topk_wT, hist32, x)


def _moe_body(be_ref, bx_ref, x_ref, w1_ref, w2_ref, ws_ref, y_ref):
    x = x_ref[...]
    gu = lax.dot_general(x, w1_ref[0], (((1,), (1,)), ((), ())),
                         preferred_element_type=jnp.float32)
    g = gu[:, :I]
    u = gu[:, I:]
    h = g * jax.nn.sigmoid(g) * u
    y_ref[...] = ws_ref[...] * lax.dot_general(
        h, w2_ref[0], (((1,), (1,)), ((), ())),
        preferred_element_type=jnp.float32)


def _grouped_mlp(x_sorted, w1, w2, w_sorted, block_expert, bx):
    grid_spec = pltpu.PrefetchScalarGridSpec(
        num_scalar_prefetch=2,
        grid=(NB,),
        in_specs=[
            pl.BlockSpec((TB, D), lambda i, be, bx: (bx[i], 0)),
            pl.BlockSpec((1, 2 * I, D), lambda i, be, bx: (be[i], 0, 0)),
            pl.BlockSpec((1, D, I), lambda i, be, bx: (be[i], 0, 0)),
            pl.BlockSpec((TB, 1), lambda i, be, bx: (bx[i], 0)),
        ],
        out_specs=pl.BlockSpec((TB, D), lambda i, be, bx: (bx[i], 0)),
    )
    return pl.pallas_call(
        _moe_body,
        grid_spec=grid_spec,
        out_shape=jax.ShapeDtypeStruct((NROWS, D), jnp.float32),
    )(block_expert, bx, x_sorted, w1, w2, w_sorted.reshape(NROWS, 1))


def _shared_body(x_ref, w1_ref, w2_ref, y_ref):
    x = x_ref[...]
    gu = lax.dot_general(x, w1_ref[...], (((1,), (1,)), ((), ())),
                         preferred_element_type=jnp.float32)
    g = gu[:, :I]
    u = gu[:, I:]
    h = g * jax.nn.sigmoid(g) * u
    y_ref[...] = lax.dot_general(h, w2_ref[...], (((1,), (1,)), ((), ())),
                                 preferred_element_type=jnp.float32)


def _shared_mlp(x, shared_w1, shared_w2):
    TBS = 512
    return pl.pallas_call(
        _shared_body,
        grid=(T // TBS,),
        in_specs=[
            pl.BlockSpec((TBS, D), lambda i: (i, 0)),
            pl.BlockSpec((2 * I, D), lambda i: (0, 0)),
            pl.BlockSpec((D, I), lambda i: (0, 0)),
        ],
        out_specs=pl.BlockSpec((TBS, D), lambda i: (i, 0)),
        out_shape=jax.ShapeDtypeStruct((T, D), jnp.float32),
    )(x, shared_w1, shared_w2)


def _combine_sc(pos, y_sorted, shared_out):
    mesh = plsc.VectorSubcoreMesh(core_axis_name="c", subcore_axis_name="s")
    CH = 32

    @functools.partial(
        pl.kernel, mesh=mesh,
        out_type=jax.ShapeDtypeStruct((T, D), jnp.float32),
        scratch_types=[
            pltpu.VMEM((K * TPW // 64, 64), jnp.int32),
            pltpu.VMEM((CH, D), jnp.float32),
            pltpu.VMEM((CH, D), jnp.float32),
            pltpu.VMEM((CH, D), jnp.float32),
            pltpu.SemaphoreType.DMA,
            pltpu.SemaphoreType.DMA,
        ],
    )
    def k(pos_hbm, ys_hbm, sh_hbm, out_hbm, posv, buf0, buf1, acc,
          sem0, sem1):
        wid = lax.axis_index("s") * 2 + lax.axis_index("c")
        pltpu.sync_copy(pos_hbm.at[wid], posv)
        for c in range(TPW // CH):
            base = wid * TPW + c * CH
            i0 = posv.at[c // 2, pl.ds((c % 2) * CH, CH)]
            i1 = posv.at[2 + c // 2, pl.ds((c % 2) * CH, CH)]
            cp0 = pltpu.async_copy(ys_hbm.at[i0], buf0, sem0)
            cp1 = pltpu.async_copy(ys_hbm.at[i1], buf1, sem1)
            pltpu.sync_copy(sh_hbm.at[pl.ds(base, CH)], acc)
            cp0.wait()
            cp1.wait()

            def jbody(j, _):
                def vbody(v, _):
                    for u in range(4):
                        sl = pl.ds(v * 64 + u * 16, 16)
                        acc[j, sl] = acc[j, sl] + buf0[j, sl] + buf1[j, sl]
                    return 0
                lax.fori_loop(0, D // 64, vbody, 0)
                return 0
            lax.fori_loop(0, CH, jbody, 0)
            pltpu.sync_copy(acc, out_hbm.at[pl.ds(base, CH)])

    return k(pos, y_sorted, shared_out)


def kernel(hidden_states, gate_w, e_score_correction_bias, w1, w2,
           shared_w1, shared_w2):
    x = hidden_states
    topk_idx, topk_wT, hist = _gating(x, gate_w, e_score_correction_bias)
    pos, x_sorted, w_sorted, block_expert, bx = _dispatch_sc(
        topk_idx, topk_wT, hist.reshape(NW, E), x)
    y_sorted = _grouped_mlp(x_sorted, w1, w2, w_sorted, block_expert, bx)
    shared_out = _shared_mlp(x, shared_w1, shared_w2)
    return _combine_sc(pos, y_sorted, shared_out)
